# Initial kernel scaffold; baseline (speedup 1.0000x reference)
import functools

import jax
import jax.numpy as jnp
from jax import lax
from jax.experimental import pallas as pl
from jax.experimental.pallas import tpu as pltpu
from jax.experimental.pallas import tpu_sc as plsc

B = 4096 * 200
F = 18
D = 32
OUT_D = 80
NW = 32
ROWS_PER_W = B // NW
CHUNK = 1280
NCHUNK = ROWS_PER_W // CHUNK


def _make_sc_kernel():
    mesh = plsc.VectorSubcoreMesh(core_axis_name="c", subcore_axis_name="s")

    @functools.partial(
        pl.kernel,
        mesh=mesh,
        out_type=jax.ShapeDtypeStruct((B, OUT_D), jnp.float32),
        scratch_types=[
            pltpu.VMEM((CHUNK,), jnp.int32),
            pltpu.VMEM((CHUNK,), jnp.int32),
            pltpu.VMEM((CHUNK, OUT_D), jnp.float32),
            pltpu.SemaphoreType.DMA,
            pltpu.SemaphoreType.DMA,
        ],
    )
    def sc_kernel(act_idx_hbm, res_idx_hbm, data_hbm, act_tab_hbm,
                  res_tab_hbm, out_hbm, ai_v, ri_v, out_v, sem_a, sem_r):
        wid = lax.axis_index("s") * 2 + lax.axis_index("c")

        def chunk_body(c, carry):
            base = wid * ROWS_PER_W + c * CHUNK
            pltpu.sync_copy(act_idx_hbm.at[pl.ds(base, CHUNK)], ai_v)
            pltpu.sync_copy(res_idx_hbm.at[pl.ds(base, CHUNK)], ri_v)
            cp_a = pltpu.make_async_copy(
                act_tab_hbm.at[ai_v], out_v.at[:, pl.ds(0, D)], sem_a)
            cp_a.start()
            cp_r = pltpu.make_async_copy(
                res_tab_hbm.at[ri_v], out_v.at[:, pl.ds(D, D)], sem_r)
            cp_r.start()
            pltpu.sync_copy(
                data_hbm.at[pl.ds(base, CHUNK), pl.ds(2, F - 2)],
                out_v.at[:, pl.ds(2 * D, F - 2)])
            cp_a.wait()
            cp_r.wait()
            pltpu.sync_copy(out_v, out_hbm.at[pl.ds(base, CHUNK)])
            return carry

        lax.fori_loop(0, NCHUNK, chunk_body, 0)

    return sc_kernel


_sc_kernel = _make_sc_kernel()


def kernel(data, act_table, res_table):
    data2 = data.reshape(B, F)
    act_idx = data2[:, 0].astype(jnp.int32)
    res_idx = data2[:, 1].astype(jnp.int32)
    out = _sc_kernel(act_idx, res_idx, data2, act_table, res_table)
    return out.reshape(4096, 200, OUT_D)


# speedup vs baseline: 1.2783x; 1.2783x over previous
import functools

import jax
import jax.numpy as jnp
from jax import lax
from jax.experimental import pallas as pl
from jax.experimental.pallas import tpu as pltpu
from jax.experimental.pallas import tpu_sc as plsc

B = 4096 * 200
F = 18
D = 32
OUT_D = 80
NW = 32
ROWS_PER_W = B // NW
CHUNK = 640
NCHUNK = ROWS_PER_W // CHUNK


def _make_sc_kernel():
    mesh = plsc.VectorSubcoreMesh(core_axis_name="c", subcore_axis_name="s")

    @functools.partial(
        pl.kernel,
        mesh=mesh,
        compiler_params=pltpu.CompilerParams(use_tc_tiling_on_sc=False),
        out_type=jax.ShapeDtypeStruct((B, OUT_D), jnp.float32),
        scratch_types=[
            pltpu.VMEM((CHUNK,), jnp.int32),
            pltpu.VMEM((CHUNK,), jnp.int32),
            pltpu.VMEM((CHUNK, D), jnp.float32),
            pltpu.VMEM((CHUNK, D), jnp.float32),
            pltpu.VMEM((CHUNK, F), jnp.float32),
            pltpu.VMEM((CHUNK, OUT_D), jnp.float32),
            pltpu.SemaphoreType.DMA,
            pltpu.SemaphoreType.DMA,
        ],
    )
    def sc_kernel(act_idx_hbm, res_idx_hbm, data_hbm, act_tab_hbm,
                  res_tab_hbm, out_hbm, ai_v, ri_v, a_v, r_v, d_v, out_v,
                  sem_a, sem_r):
        wid = lax.axis_index("s") * 2 + lax.axis_index("c")
        L = 16

        def chunk_body(c, carry):
            base = wid * ROWS_PER_W + c * CHUNK
            pltpu.sync_copy(act_idx_hbm.at[pl.ds(base, CHUNK)], ai_v)
            pltpu.sync_copy(res_idx_hbm.at[pl.ds(base, CHUNK)], ri_v)
            cp_a = pltpu.make_async_copy(act_tab_hbm.at[ai_v], a_v, sem_a)
            cp_a.start()
            cp_r = pltpu.make_async_copy(res_tab_hbm.at[ri_v], r_v, sem_r)
            cp_r.start()
            pltpu.sync_copy(data_hbm.at[pl.ds(base, CHUNK)], d_v)
            cp_a.wait()
            cp_r.wait()

            def row_body(i, rcarry):
                out_v[i, pl.ds(0, L)] = a_v[i, pl.ds(0, L)]
                out_v[i, pl.ds(L, L)] = a_v[i, pl.ds(L, L)]
                out_v[i, pl.ds(2 * L, L)] = r_v[i, pl.ds(0, L)]
                out_v[i, pl.ds(3 * L, L)] = r_v[i, pl.ds(L, L)]
                out_v[i, pl.ds(4 * L, L)] = d_v[i, pl.ds(2, L)]
                return rcarry

            lax.fori_loop(0, CHUNK, row_body, 0)
            pltpu.sync_copy(out_v, out_hbm.at[pl.ds(base, CHUNK)])
            return carry

        lax.fori_loop(0, NCHUNK, chunk_body, 0)

    return sc_kernel


_sc_kernel = _make_sc_kernel()


def kernel(data, act_table, res_table):
    data2 = data.reshape(B, F)
    act_idx = data2[:, 0].astype(jnp.int32)
    res_idx = data2[:, 1].astype(jnp.int32)
    out = _sc_kernel(act_idx, res_idx, data2, act_table, res_table)
    return out.reshape(4096, 200, OUT_D)


# speedup vs baseline: 1.6537x; 1.2937x over previous
import functools

import jax
import jax.numpy as jnp
from jax import lax
from jax.experimental import pallas as pl
from jax.experimental.pallas import tpu as pltpu
from jax.experimental.pallas import tpu_sc as plsc

N_ACT = 1000001
N_RES = 100001
B = 4096 * 200
F = 18
D = 32
OUT_D = 80
NW = 32
ROWS_PER_W = B // NW
CHUNK = 640
NCHUNK = ROWS_PER_W // CHUNK
L = 16


def _make_sc_kernel():
    mesh = plsc.VectorSubcoreMesh(core_axis_name="c", subcore_axis_name="s")

    @functools.partial(
        pl.kernel,
        mesh=mesh,
        compiler_params=pltpu.CompilerParams(
            use_tc_tiling_on_sc=False, needs_layout_passes=False),
        out_type=jax.ShapeDtypeStruct((B, OUT_D), jnp.float32),
        scratch_types=[
            pltpu.VMEM((CHUNK * F,), jnp.float32),
            pltpu.VMEM((CHUNK,), jnp.int32),
            pltpu.VMEM((CHUNK,), jnp.int32),
            pltpu.VMEM((CHUNK, D), jnp.float32),
            pltpu.VMEM((CHUNK, D), jnp.float32),
            pltpu.VMEM((CHUNK, OUT_D), jnp.float32),
            pltpu.SemaphoreType.DMA,
            pltpu.SemaphoreType.DMA,
        ],
    )
    def sc_kernel(data_hbm, act2, res2, out_hbm,
                  d_v, ai_v, ri_v, a_v, r_v, out_v, sem_a, sem_r):
        wid = lax.axis_index("s") * 2 + lax.axis_index("c")

        def chunk_body(c, carry):
            base = wid * ROWS_PER_W + c * CHUNK
            pltpu.sync_copy(data_hbm.at[pl.ds(base * F, CHUNK * F)], d_v)

            def idx_body(j, icarry):
                offs = (lax.iota(jnp.int32, L) + j * L) * F
                av = plsc.load_gather(d_v, [offs])
                rv = plsc.load_gather(d_v, [offs + 1])
                ai_v[pl.ds(j * L, L)] = av.astype(jnp.int32)
                ri_v[pl.ds(j * L, L)] = rv.astype(jnp.int32)
                return icarry

            lax.fori_loop(0, CHUNK // L, idx_body, 0)

            cp_a = pltpu.make_async_copy(act2.at[ai_v], a_v, sem_a)
            cp_a.start()
            cp_r = pltpu.make_async_copy(res2.at[ri_v], r_v, sem_r)
            cp_r.start()

            def feat_body(i, fcarry):
                out_v[i, pl.ds(4 * L, L)] = d_v[pl.ds(i * F + 2, L)]
                return fcarry

            lax.fori_loop(0, CHUNK, feat_body, 0)
            cp_a.wait()
            cp_r.wait()

            def row_body(i, rcarry):
                out_v[i, pl.ds(0, L)] = a_v[i, pl.ds(0, L)]
                out_v[i, pl.ds(L, L)] = a_v[i, pl.ds(L, L)]
                out_v[i, pl.ds(2 * L, L)] = r_v[i, pl.ds(0, L)]
                out_v[i, pl.ds(3 * L, L)] = r_v[i, pl.ds(L, L)]
                return rcarry

            lax.fori_loop(0, CHUNK, row_body, 0)
            pltpu.sync_copy(out_v, out_hbm.at[pl.ds(base, CHUNK)])
            return carry

        lax.fori_loop(0, NCHUNK, chunk_body, 0)

    return sc_kernel


_sc_kernel = _make_sc_kernel()


def kernel(data, act_table, res_table):
    data_flat = data.reshape(B * F)
    out = _sc_kernel(data_flat, act_table, res_table)
    return out.reshape(4096, 200, OUT_D)


# speedup vs baseline: 2.2816x; 1.3796x over previous
import functools

import jax
import jax.numpy as jnp
from jax import lax
from jax.experimental import pallas as pl
from jax.experimental.pallas import tpu as pltpu
from jax.experimental.pallas import tpu_sc as plsc

N_ACT = 1000001
N_RES = 100001
NB = 4096
NS = 200
B = NB * NS
F = 18
D = 32
OUT_D = 80
NW = 32
B_PER_W = NB // NW
BC = 16
NCHUNK = B_PER_W // BC
CHUNK = BC * NS
SUB = 800
NSUB = CHUNK // SUB
JPS = SUB // NS
L = 16


def _make_sc_kernel():
    mesh = plsc.VectorSubcoreMesh(core_axis_name="c", subcore_axis_name="s")

    @functools.partial(
        pl.kernel,
        mesh=mesh,
        compiler_params=pltpu.CompilerParams(
            use_tc_tiling_on_sc=False, needs_layout_passes=False),
        out_type=jax.ShapeDtypeStruct((B, OUT_D), jnp.float32),
        scratch_types=[
            pltpu.VMEM((2, NS, BC), jnp.float32),
            pltpu.VMEM((F - 2, NS, BC), jnp.float32),
            pltpu.VMEM((CHUNK,), jnp.int32),
            pltpu.VMEM((CHUNK,), jnp.int32),
            pltpu.VMEM((SUB, D), jnp.float32),
            pltpu.VMEM((SUB, D), jnp.float32),
            pltpu.VMEM((SUB, F - 2), jnp.float32),
            pltpu.SemaphoreType.DMA,
            pltpu.SemaphoreType.DMA,
        ],
    )
    def sc_kernel(dataT, act2, res2, out_hbm,
                  id_v, f_v, ai_v, ri_v, a_v, r_v, f2_v, sem_a, sem_r):
        wid = lax.axis_index("s") * 2 + lax.axis_index("c")
        lanes = lax.iota(jnp.int32, L)

        def chunk_body(cj, carry):
            b0 = wid * B_PER_W + cj * BC
            r0 = b0 * NS
            pltpu.sync_copy(dataT.at[pl.ds(0, 2), :, pl.ds(b0, BC)], id_v)
            pltpu.sync_copy(dataT.at[pl.ds(2, F - 2), :, pl.ds(b0, BC)], f_v)

            def idx_body(s, icarry):
                pos = lanes * NS + s
                va = id_v[0, s, pl.ds(0, L)].astype(jnp.int32)
                vr = id_v[1, s, pl.ds(0, L)].astype(jnp.int32)
                plsc.store_scatter(ai_v, [pos], va)
                plsc.store_scatter(ri_v, [pos], vr)
                return icarry

            lax.fori_loop(0, NS, idx_body, 0, unroll=2)

            def sub_body(k, scarry):
                cp_a = pltpu.make_async_copy(
                    act2.at[ai_v.at[pl.ds(k * SUB, SUB)]], a_v, sem_a)
                cp_a.start()
                cp_r = pltpu.make_async_copy(
                    res2.at[ri_v.at[pl.ds(k * SUB, SUB)]], r_v, sem_r)
                cp_r.start()

                def feat_body(i, fcarry):
                    j2 = i // NS
                    s = i % NS
                    g = plsc.load_gather(
                        f_v, [lanes,
                              jnp.full((L,), s, jnp.int32),
                              jnp.full((L,), k * JPS + j2, jnp.int32)])
                    f2_v[i, pl.ds(0, L)] = g
                    return fcarry

                lax.fori_loop(0, SUB, feat_body, 0, unroll=4)
                cp_a.wait()
                cp_r.wait()
                rk = r0 + k * SUB
                pltpu.sync_copy(a_v, out_hbm.at[pl.ds(rk, SUB), pl.ds(0, D)])
                pltpu.sync_copy(r_v, out_hbm.at[pl.ds(rk, SUB), pl.ds(D, D)])
                pltpu.sync_copy(
                    f2_v, out_hbm.at[pl.ds(rk, SUB), pl.ds(2 * D, F - 2)])
                return scarry

            lax.fori_loop(0, NSUB, sub_body, 0)
            return carry

        lax.fori_loop(0, NCHUNK, chunk_body, 0)

    return sc_kernel


_sc_kernel = _make_sc_kernel()


def kernel(data, act_table, res_table):
    dataT = data.transpose(2, 1, 0)
    out = _sc_kernel(dataT, act_table, res_table)
    return out.reshape(NB, NS, OUT_D)
